# baseline (speedup 1.0000x reference)
import functools

import jax
import jax.numpy as jnp
from jax import lax
from jax.experimental import pallas as pl
from jax.experimental.pallas import tpu as pltpu
from jax.experimental.pallas import tpu_sc as plsc

NUM_ENTITY = 1000000
DIM = 64
BATCH = 16384

NC = 2
NS = 16
L = 16
NW = NC * NS
B_PER_W = BATCH // NW
IDX_CHUNK = 128
N_CHUNKS = B_PER_W // IDX_CHUNK
PAIR_ROWS = NUM_ENTITY // 2


def _make_sc_kernel():
    mesh = plsc.VectorSubcoreMesh(core_axis_name="c", subcore_axis_name="s")

    @functools.partial(
        pl.kernel,
        out_type=jax.ShapeDtypeStruct((BATCH,), jnp.float32),
        mesh=mesh,
        compiler_params=pltpu.CompilerParams(
            needs_layout_passes=False, use_tc_tiling_on_sc=False),
        scratch_types=[
            pltpu.VMEM((2, N_CHUNKS, IDX_CHUNK), jnp.int32),
            pltpu.VMEM((2, N_CHUNKS, IDX_CHUNK), jnp.int32),
            pltpu.VMEM((2, IDX_CHUNK, 2 * DIM), jnp.bfloat16),
            pltpu.VMEM((2, IDX_CHUNK, 2 * DIM), jnp.bfloat16),
            pltpu.VMEM((B_PER_W,), jnp.float32),
            pltpu.SemaphoreType.DMA,
            pltpu.SemaphoreType.DMA,
        ],
    )
    def sc_kernel(idx_hbm, t1_hbm, t2_hbm, out_hbm,
                  idx_v, pair_v, e1_v, e2_v, out_v, sem1, sem2):
        wid = lax.axis_index("s") * NC + lax.axis_index("c")

        pltpu.sync_copy(idx_hbm.at[wid], idx_v)
        pltpu.sync_copy(idx_hbm.at[wid], pair_v)

        for t in range(2):
            for c in range(N_CHUNKS):
                for j in range(IDX_CHUNK // L):
                    sl = pl.ds(j * L, L)
                    pair_v[t, c, sl] = pair_v[t, c, sl] >> 1

        lanes = lax.iota(jnp.int32, L)

        def fire(c):
            b = c % 2
            return (pltpu.async_copy(
                        t1_hbm.at[pair_v.at[0, c]], e1_v.at[b], sem1),
                    pltpu.async_copy(
                        t2_hbm.at[pair_v.at[1, c]], e2_v.at[b], sem2))

        pending = fire(0)
        for c in range(N_CHUNKS):
            nxt = fire(c + 1) if c + 1 < N_CHUNKS else None
            pending[0].wait()
            pending[1].wait()
            b = c % 2

            def compute(g, _, c=c, b=b):
                base = g * L
                par1 = (idx_v[0, c, pl.ds(base, L)] & 1) * DIM
                par2 = (idx_v[1, c, pl.ds(base, L)] & 1) * DIM
                row_vec = jnp.zeros((L,), jnp.float32)
                for k in range(L):
                    i = base + k
                    off1 = par1[k]
                    off2 = par2[k]
                    acc = jnp.zeros((L,), jnp.float32)
                    for cc in range(DIM // (2 * L)):
                        a = e1_v[b, i, pl.ds(off1 + cc * 2 * L, 2 * L)]
                        bb = e2_v[b, i, pl.ds(off2 + cc * 2 * L, 2 * L)]
                        a0, a1 = plsc.unpack(a, format=plsc.PackFormat.INTERLEAVED)
                        b0, b1 = plsc.unpack(bb, format=plsc.PackFormat.INTERLEAVED)
                        d0 = a0 - b0
                        d1 = a1 - b1
                        acc = acc + d0 * d0 + d1 * d1
                    row_vec = jnp.where(lanes == k, -jnp.sum(acc), row_vec)
                out_v[pl.ds(c * IDX_CHUNK + base, L)] = row_vec
                return _

            lax.fori_loop(0, IDX_CHUNK // L, compute, 0)
            pending = nxt

        pltpu.sync_copy(out_v, out_hbm.at[pl.ds(wid * B_PER_W, B_PER_W)])

    return sc_kernel


_sc_kernel = _make_sc_kernel()


@jax.jit
def kernel(idxs, emb_in, emb_out):
    t1 = emb_in.astype(jnp.bfloat16).reshape(PAIR_ROWS, 2 * DIM)
    t2 = emb_out.astype(jnp.bfloat16).reshape(PAIR_ROWS, 2 * DIM)
    idx_w = (idxs.reshape(NW, B_PER_W, 2)
             .transpose(0, 2, 1)
             .reshape(NW, 2, N_CHUNKS, IDX_CHUNK))
    return _sc_kernel(idx_w, t1, t2)


# speedup vs baseline: 1.4012x; 1.4012x over previous
import functools

import jax
import jax.numpy as jnp
from jax import lax
from jax.experimental import pallas as pl
from jax.experimental.pallas import tpu as pltpu
from jax.experimental.pallas import tpu_sc as plsc

NUM_ENTITY = 1000000
DIM = 64
BATCH = 16384

NC = 2
NS = 16
L = 16
NW = NC * NS
B_PER_W = BATCH // NW
IDX_CHUNK = 128
N_CHUNKS = B_PER_W // IDX_CHUNK
PAIR_ROWS = NUM_ENTITY // 2


def _make_sc_kernel():
    mesh = plsc.VectorSubcoreMesh(core_axis_name="c", subcore_axis_name="s")

    @functools.partial(
        pl.kernel,
        out_type=jax.ShapeDtypeStruct((BATCH,), jnp.float32),
        mesh=mesh,
        compiler_params=pltpu.CompilerParams(
            needs_layout_passes=False, use_tc_tiling_on_sc=False),
        scratch_types=[
            pltpu.VMEM((2, N_CHUNKS, IDX_CHUNK), jnp.int32),
            pltpu.VMEM((2, IDX_CHUNK, 2 * DIM), jnp.float32),
            pltpu.VMEM((2, IDX_CHUNK, 2 * DIM), jnp.float32),
            pltpu.VMEM((B_PER_W,), jnp.float32),
            pltpu.SemaphoreType.DMA,
            pltpu.SemaphoreType.DMA,
        ],
    )
    def sc_kernel(idx_hbm, t1_hbm, t2_hbm, out_hbm,
                  idx_v, e1_v, e2_v, out_v, sem1, sem2):
        wid = lax.axis_index("s") * NC + lax.axis_index("c")

        pltpu.sync_copy(idx_hbm.at[wid], idx_v)

        lanes = lax.iota(jnp.int32, L)

        def fire(c):
            b = c % 2
            return (pltpu.async_copy(
                        t1_hbm.at[idx_v.at[0, c]], e1_v.at[b], sem1),
                    pltpu.async_copy(
                        t2_hbm.at[idx_v.at[1, c]], e2_v.at[b], sem2))

        pending = fire(0)
        for c in range(N_CHUNKS):
            nxt = fire(c + 1) if c + 1 < N_CHUNKS else None
            pending[0].wait()
            pending[1].wait()
            b = c % 2

            def compute(g, _, c=c, b=b):
                base = g * L
                row_vec = jnp.zeros((L,), jnp.float32)
                for k in range(L):
                    i = base + k
                    acc = jnp.zeros((L,), jnp.float32)
                    for cc in range(DIM // L):
                        a = e1_v[b, i, pl.ds(cc * L, L)]
                        bb = e2_v[b, i, pl.ds(cc * L, L)]
                        diff = a - bb
                        acc = acc + diff * diff
                    row_vec = jnp.where(lanes == k, -jnp.sum(acc), row_vec)
                out_v[pl.ds(c * IDX_CHUNK + base, L)] = row_vec
                return _

            lax.fori_loop(0, IDX_CHUNK // L, compute, 0)
            pending = nxt

        pltpu.sync_copy(out_v, out_hbm.at[pl.ds(wid * B_PER_W, B_PER_W)])

    return sc_kernel


_sc_kernel = _make_sc_kernel()


@jax.jit
def kernel(idxs, emb_in, emb_out):
    t1 = jnp.pad(emb_in, ((0, 0), (0, DIM)))
    t2 = jnp.pad(emb_out, ((0, 0), (0, DIM)))
    idx_w = (idxs.reshape(NW, B_PER_W, 2)
             .transpose(0, 2, 1)
             .reshape(NW, 2, N_CHUNKS, IDX_CHUNK))
    return _sc_kernel(idx_w, t1, t2)


# speedup vs baseline: 1.4034x; 1.0015x over previous
import functools

import jax
import jax.numpy as jnp
from jax import lax
from jax.experimental import pallas as pl
from jax.experimental.pallas import tpu as pltpu
from jax.experimental.pallas import tpu_sc as plsc

NUM_ENTITY = 1000000
DIM = 64
BATCH = 16384

NC = 2
NS = 16
L = 16
NW = NC * NS
B_PER_W = BATCH // NW
IDX_CHUNK = 128
N_CHUNKS = B_PER_W // IDX_CHUNK
PAIR_ROWS = NUM_ENTITY // 2


def _make_sc_kernel():
    mesh = plsc.VectorSubcoreMesh(core_axis_name="c", subcore_axis_name="s")

    @functools.partial(
        pl.kernel,
        out_type=jax.ShapeDtypeStruct((BATCH,), jnp.float32),
        mesh=mesh,
        compiler_params=pltpu.CompilerParams(
            needs_layout_passes=False, use_tc_tiling_on_sc=False),
        scratch_types=[
            pltpu.VMEM((2, N_CHUNKS, IDX_CHUNK), jnp.int32),
            pltpu.VMEM((2, IDX_CHUNK, 2 * DIM), jnp.float32),
            pltpu.VMEM((2, IDX_CHUNK, 2 * DIM), jnp.float32),
            pltpu.VMEM((B_PER_W,), jnp.float32),
            pltpu.SemaphoreType.DMA,
            pltpu.SemaphoreType.DMA,
        ],
    )
    def sc_kernel(idx_hbm, t1_hbm, t2_hbm, out_hbm,
                  idx_v, e1_v, e2_v, out_v, sem1, sem2):
        wid = lax.axis_index("s") * NC + lax.axis_index("c")

        pltpu.sync_copy(idx_hbm.at[wid], idx_v)

        lanes = lax.iota(jnp.int32, L)

        def fire(c):
            b = c % 2
            return (pltpu.async_copy(
                        t1_hbm.at[idx_v.at[0, c]], e1_v.at[b], sem1),
                    pltpu.async_copy(
                        t2_hbm.at[idx_v.at[1, c]], e2_v.at[b], sem2))

        pending = fire(0)
        for c in range(N_CHUNKS):
            nxt = fire(c + 1) if c + 1 < N_CHUNKS else None
            pending[0].wait()
            pending[1].wait()
            b = c % 2

            def compute(g, _, c=c, b=b):
                base = g * L
                row_vec = jnp.zeros((L,), jnp.float32)
                for k in range(L):
                    i = base + k
                    acc = jnp.zeros((L,), jnp.float32)
                    for cc in range(DIM // L):
                        a = e1_v[b, i, pl.ds(cc * L, L)]
                        bb = e2_v[b, i, pl.ds(cc * L, L)]
                        diff = a - bb
                        acc = acc + diff * diff
                    row_vec = jnp.where(lanes == k, -jnp.sum(acc), row_vec)
                out_v[pl.ds(c * IDX_CHUNK + base, L)] = row_vec
                return _

            lax.fori_loop(0, IDX_CHUNK // L, compute, 0)
            pending = nxt

        pltpu.sync_copy(out_v, out_hbm.at[pl.ds(wid * B_PER_W, B_PER_W)])

    return sc_kernel


_sc_kernel = _make_sc_kernel()


@jax.jit
def kernel(idxs, emb_in, emb_out):
    t1 = jnp.pad(emb_in, ((0, 0), (0, DIM)))
    t2 = jnp.pad(emb_out, ((0, 0), (0, DIM)))
    t1, t2 = jax.lax.optimization_barrier((t1, t2))
    idx_w = (idxs.reshape(NW, B_PER_W, 2)
             .transpose(0, 2, 1)
             .reshape(NW, 2, N_CHUNKS, IDX_CHUNK))
    return _sc_kernel(idx_w, t1, t2)
